# baseline (speedup 1.0000x reference)
import jax
import jax.numpy as jnp
from jax.experimental import pallas as pl
from jax.experimental.pallas import tpu as pltpu

N_HALF = 25000
D = 256
BC = 1000
NCH = N_HALF // BC
NC = 2 * NCH
NBUF = 6


def _mm_kernel(ctx_hbm, smp_hbm, w_ref, b_ref, out_hbm, xbuf, obuf, in_sem, out_sem):
    def start_in(c, slot):
        j = c // 2

        @pl.when(c % 2 == 0)
        def _():
            pltpu.make_async_copy(
                ctx_hbm.at[pl.ds(j * BC, BC), :], xbuf.at[slot], in_sem.at[slot]
            ).start()

        @pl.when(c % 2 == 1)
        def _():
            pltpu.make_async_copy(
                smp_hbm.at[pl.ds(j * BC, BC), :], xbuf.at[slot], in_sem.at[slot]
            ).start()

    def wait_in(slot):
        pltpu.make_async_copy(
            ctx_hbm.at[pl.ds(0, BC), :], xbuf.at[slot], in_sem.at[slot]
        ).wait()

    def out_off(c):
        return (c % 2) * N_HALF + (c // 2) * BC

    def start_out(c, slot):
        pltpu.make_async_copy(
            obuf.at[slot], out_hbm.at[pl.ds(out_off(c), BC), :], out_sem.at[slot]
        ).start()

    def wait_out(c, slot):
        pltpu.make_async_copy(
            obuf.at[slot], out_hbm.at[pl.ds(out_off(c), BC), :], out_sem.at[slot]
        ).wait()

    for s in range(NBUF):
        start_in(s, s)

    def body(c, carry):
        slot = jax.lax.rem(c, NBUF)

        @pl.when(c >= NBUF)
        def _():
            wait_out(c - NBUF, slot)

        wait_in(slot)
        obuf[slot] = (
            jnp.dot(xbuf[slot], w_ref[...], preferred_element_type=jnp.float32)
            + b_ref[...]
        )
        start_out(c, slot)

        @pl.when(c + NBUF < NC)
        def _():
            start_in(c + NBUF, slot)

        return carry

    jax.lax.fori_loop(0, NC, body, 0)

    for k in range(NC - NBUF, NC):
        wait_out(k, k % NBUF)


def kernel(context, sample, W_proj, b_proj):
    b2d = b_proj.reshape(1, D)
    out = pl.pallas_call(
        _mm_kernel,
        in_specs=[
            pl.BlockSpec(memory_space=pl.ANY),
            pl.BlockSpec(memory_space=pl.ANY),
            pl.BlockSpec(memory_space=pltpu.VMEM),
            pl.BlockSpec(memory_space=pltpu.VMEM),
        ],
        out_specs=pl.BlockSpec(memory_space=pl.ANY),
        out_shape=jax.ShapeDtypeStruct((2 * N_HALF, D), jnp.float32),
        scratch_shapes=[
            pltpu.VMEM((NBUF, BC, D), jnp.float32),
            pltpu.VMEM((NBUF, BC, D), jnp.float32),
            pltpu.SemaphoreType.DMA((NBUF,)),
            pltpu.SemaphoreType.DMA((NBUF,)),
        ],
    )(context, sample, W_proj, b2d)
    return out


# speedup vs baseline: 1.0174x; 1.0174x over previous
import jax
import jax.numpy as jnp
from jax.experimental import pallas as pl
from jax.experimental.pallas import tpu as pltpu

N_HALF = 25000
D = 256
BC = 5000
NCH = N_HALF // BC
NC = 2 * NCH
NBUF = 3


def _mm_kernel(ctx_hbm, smp_hbm, w_ref, b_ref, out_hbm, xbuf, obuf, in_sem, out_sem):
    def start_in(c, slot):
        j = c // 2

        @pl.when(c % 2 == 0)
        def _():
            pltpu.make_async_copy(
                ctx_hbm.at[pl.ds(j * BC, BC), :], xbuf.at[slot], in_sem.at[slot]
            ).start()

        @pl.when(c % 2 == 1)
        def _():
            pltpu.make_async_copy(
                smp_hbm.at[pl.ds(j * BC, BC), :], xbuf.at[slot], in_sem.at[slot]
            ).start()

    def wait_in(slot):
        pltpu.make_async_copy(
            ctx_hbm.at[pl.ds(0, BC), :], xbuf.at[slot], in_sem.at[slot]
        ).wait()

    def out_off(c):
        return (c % 2) * N_HALF + (c // 2) * BC

    def start_out(c, slot):
        pltpu.make_async_copy(
            obuf.at[slot], out_hbm.at[pl.ds(out_off(c), BC), :], out_sem.at[slot]
        ).start()

    def wait_out(c, slot):
        pltpu.make_async_copy(
            obuf.at[slot], out_hbm.at[pl.ds(out_off(c), BC), :], out_sem.at[slot]
        ).wait()

    for s in range(NBUF):
        start_in(s, s)

    def body(c, carry):
        slot = jax.lax.rem(c, NBUF)

        @pl.when(c >= NBUF)
        def _():
            wait_out(c - NBUF, slot)

        wait_in(slot)
        obuf[slot] = (
            jnp.dot(xbuf[slot], w_ref[...], preferred_element_type=jnp.float32)
            + b_ref[...]
        )
        start_out(c, slot)

        @pl.when(c + NBUF < NC)
        def _():
            start_in(c + NBUF, slot)

        return carry

    jax.lax.fori_loop(0, NC, body, 0)

    for k in range(NC - NBUF, NC):
        wait_out(k, k % NBUF)


def kernel(context, sample, W_proj, b_proj):
    b2d = b_proj.reshape(1, D)
    out = pl.pallas_call(
        _mm_kernel,
        in_specs=[
            pl.BlockSpec(memory_space=pl.ANY),
            pl.BlockSpec(memory_space=pl.ANY),
            pl.BlockSpec(memory_space=pltpu.VMEM),
            pl.BlockSpec(memory_space=pltpu.VMEM),
        ],
        out_specs=pl.BlockSpec(memory_space=pl.ANY),
        out_shape=jax.ShapeDtypeStruct((2 * N_HALF, D), jnp.float32),
        scratch_shapes=[
            pltpu.VMEM((NBUF, BC, D), jnp.float32),
            pltpu.VMEM((NBUF, BC, D), jnp.float32),
            pltpu.SemaphoreType.DMA((NBUF,)),
            pltpu.SemaphoreType.DMA((NBUF,)),
        ],
    )(context, sample, W_proj, b2d)
    return out


# speedup vs baseline: 1.0184x; 1.0009x over previous
import jax
import jax.numpy as jnp
from jax.experimental import pallas as pl
from jax.experimental.pallas import tpu as pltpu

N_HALF = 25000
D = 256
BC = 5000
NCH = N_HALF // BC
NC = 2 * NCH
NBUF = 4


def _mm_kernel(ctx_hbm, smp_hbm, w_ref, b_ref, out_hbm, xbuf, obuf, in_sem, out_sem):
    def start_in(c, slot):
        j = c // 2

        @pl.when(c % 2 == 0)
        def _():
            pltpu.make_async_copy(
                ctx_hbm.at[pl.ds(j * BC, BC), :], xbuf.at[slot], in_sem.at[slot]
            ).start()

        @pl.when(c % 2 == 1)
        def _():
            pltpu.make_async_copy(
                smp_hbm.at[pl.ds(j * BC, BC), :], xbuf.at[slot], in_sem.at[slot]
            ).start()

    def wait_in(slot):
        pltpu.make_async_copy(
            ctx_hbm.at[pl.ds(0, BC), :], xbuf.at[slot], in_sem.at[slot]
        ).wait()

    def out_off(c):
        return (c % 2) * N_HALF + (c // 2) * BC

    def start_out(c, slot):
        pltpu.make_async_copy(
            obuf.at[slot], out_hbm.at[pl.ds(out_off(c), BC), :], out_sem.at[slot]
        ).start()

    def wait_out(c, slot):
        pltpu.make_async_copy(
            obuf.at[slot], out_hbm.at[pl.ds(out_off(c), BC), :], out_sem.at[slot]
        ).wait()

    for s in range(NBUF):
        start_in(s, s)

    def body(c, carry):
        slot = jax.lax.rem(c, NBUF)

        @pl.when(c >= NBUF)
        def _():
            wait_out(c - NBUF, slot)

        wait_in(slot)
        obuf[slot] = (
            jnp.dot(xbuf[slot], w_ref[...], preferred_element_type=jnp.float32)
            + b_ref[...]
        )
        start_out(c, slot)

        @pl.when(c + NBUF < NC)
        def _():
            start_in(c + NBUF, slot)

        return carry

    jax.lax.fori_loop(0, NC, body, 0)

    for k in range(NC - NBUF, NC):
        wait_out(k, k % NBUF)


def kernel(context, sample, W_proj, b_proj):
    b2d = b_proj.reshape(1, D)
    out = pl.pallas_call(
        _mm_kernel,
        in_specs=[
            pl.BlockSpec(memory_space=pl.ANY),
            pl.BlockSpec(memory_space=pl.ANY),
            pl.BlockSpec(memory_space=pltpu.VMEM),
            pl.BlockSpec(memory_space=pltpu.VMEM),
        ],
        out_specs=pl.BlockSpec(memory_space=pl.ANY),
        out_shape=jax.ShapeDtypeStruct((2 * N_HALF, D), jnp.float32),
        scratch_shapes=[
            pltpu.VMEM((NBUF, BC, D), jnp.float32),
            pltpu.VMEM((NBUF, BC, D), jnp.float32),
            pltpu.SemaphoreType.DMA((NBUF,)),
            pltpu.SemaphoreType.DMA((NBUF,)),
        ],
    )(context, sample, W_proj, b2d)
    return out
